# baseline (speedup 1.0000x reference)
import functools

import jax
import jax.numpy as jnp
from jax import lax
from jax.experimental import pallas as pl
from jax.experimental.pallas import tpu as pltpu
from jax.experimental.pallas import tpu_sc as plsc

B, S, D_LLM, D_VIT, K, TOPK = 4, 2048, 2048, 1024, 512, 128
TS = 256

_NC, _NS = 2, 16
_NW = _NC * _NS
_RPW = (B * TOPK) // _NW


def _route_kernel(lq_ref, vq_ref, kl_ref, kv_ref, a_ref, asel_ref, idx_ref):
    k_ratio = float(D_VIT) / float(D_LLM)
    score = lax.dot_general(lq_ref[...], kl_ref[...], (((1,), (1,)), ((), ())),
                            preferred_element_type=jnp.float32)
    score = score + k_ratio * lax.dot_general(
        vq_ref[...], kv_ref[...], (((1,), (1,)), ((), ())),
        preferred_element_type=jnp.float32)

    u = lax.bitcast_convert_type(score, jnp.int32)
    key = u ^ ((u >> 31) & jnp.int32(0x7FFFFFFF))

    def count_ge(m):
        return jnp.sum((key >= m).astype(jnp.int32), axis=1, keepdims=True)

    int_min = jnp.full((B, 1), -2147483648, jnp.int32)
    zero = jnp.zeros((B, 1), jnp.int32)
    t = jnp.where(count_ge(zero) >= TOPK, zero, int_min)
    for bit in range(30, -1, -1):
        cand = t | jnp.int32(1 << bit)
        t = jnp.where(count_ge(cand) >= TOPK, cand, t)

    gt = key > t
    eq = key == t
    need = (TOPK - jnp.sum(gt.astype(jnp.int32), axis=1, keepdims=True)
            ).astype(jnp.float32)
    rows = lax.broadcasted_iota(jnp.int32, (K, K), 0)
    cols = lax.broadcasted_iota(jnp.int32, (K, K), 1)
    tri = (rows <= cols).astype(jnp.float32)
    ecs = jnp.dot(eq.astype(jnp.float32), tri,
                  preferred_element_type=jnp.float32)
    mask = jnp.logical_or(gt, jnp.logical_and(eq, ecs <= need))

    mf = mask.astype(jnp.float32)
    pos = jnp.dot(mf, tri, preferred_element_type=jnp.float32)
    eye = (rows == cols).astype(jnp.float32)
    mf_col = lax.dot_general(eye, mf, (((1,), (1,)), ((), ())),
                             preferred_element_type=jnp.float32)
    pos_col = lax.dot_general(eye, pos, (((1,), (1,)), ((), ())),
                              preferred_element_type=jnp.float32)
    jj = lax.broadcasted_iota(jnp.int32, (K, TOPK), 1)
    kk = lax.broadcasted_iota(jnp.int32, (K, TOPK), 0)
    for b in range(B):
        slot_b = (pos_col[:, b:b + 1] - 1.0).astype(jnp.int32)
        sel_b = (slot_b == jj) & (mf_col[:, b:b + 1] > 0.5)
        p_b = jnp.where(sel_b, 1.0, 0.0)
        asel_ref[b] = jnp.dot(a_ref[...], p_b,
                              preferred_element_type=jnp.float32)
        idx_ref[b] = jnp.sum(jnp.where(sel_b, kk, 0), axis=0, keepdims=True)


def _sc_gather_body(tbl_hbm, idx_hbm, out_hbm, idx_v, rows_v, sem):
    wid = lax.axis_index("s") * _NC + lax.axis_index("c")
    base = wid * _RPW
    pltpu.sync_copy(idx_hbm.at[pl.ds(base, _RPW)], idx_v)
    pltpu.async_copy(tbl_hbm.at[idx_v], rows_v, sem).wait()
    pltpu.sync_copy(rows_v, out_hbm.at[pl.ds(base, _RPW)])


def _make_sc_gather():
    return pl.kernel(
        _sc_gather_body,
        out_type=jax.ShapeDtypeStruct((B * TOPK, D_LLM), jnp.float32),
        mesh=plsc.VectorSubcoreMesh(core_axis_name="c", subcore_axis_name="s",
                                    num_cores=_NC, num_subcores=_NS),
        scratch_types=[
            pltpu.VMEM((_RPW,), jnp.int32),
            pltpu.VMEM((_RPW, D_LLM), jnp.float32),
            pltpu.SemaphoreType.DMA,
        ],
    )


def _apply_kernel(x_ref, a_ref, b_ref, out_ref):
    xb = x_ref[0]
    t = jnp.dot(xb, a_ref[0], preferred_element_type=jnp.float32)
    out_ref[0] = xb + jnp.dot(t, b_ref[0], preferred_element_type=jnp.float32)


@jax.jit
def kernel(x, llm_query, vit_query, static_keys_llm, static_keys_vit,
           rank_A_pool, rank_B_pool):
    a_sel, idx = pl.pallas_call(
        _route_kernel,
        out_shape=[
            jax.ShapeDtypeStruct((B, D_LLM, TOPK), jnp.float32),
            jax.ShapeDtypeStruct((B, 1, TOPK), jnp.int32),
        ],
    )(llm_query, vit_query, static_keys_llm, static_keys_vit, rank_A_pool)

    b_sel = _make_sc_gather()(rank_B_pool, idx.reshape(B * TOPK))
    b_sel = b_sel.reshape(B, TOPK, D_LLM)

    out = pl.pallas_call(
        _apply_kernel,
        grid=(B, S // TS),
        in_specs=[
            pl.BlockSpec((1, TS, D_LLM), lambda b, s: (b, s, 0)),
            pl.BlockSpec((1, D_LLM, TOPK), lambda b, s: (b, 0, 0)),
            pl.BlockSpec((1, TOPK, D_LLM), lambda b, s: (b, 0, 0)),
        ],
        out_specs=pl.BlockSpec((1, TS, D_LLM), lambda b, s: (b, s, 0)),
        out_shape=jax.ShapeDtypeStruct((B, S, D_LLM), jnp.float32),
    )(x, a_sel, b_sel)
    return out


# speedup vs baseline: 1.1371x; 1.1371x over previous
import jax
import jax.numpy as jnp
from jax import lax
from jax.experimental import pallas as pl
from jax.experimental.pallas import tpu as pltpu

B, S, D_LLM, D_VIT, K, TOPK = 4, 2048, 2048, 1024, 512, 128
TS = 256


def _fused_kernel(lq_ref, vq_ref, kl_ref, kv_ref, a_ref, bp_ref, x_ref,
                  out_ref, asel_s, bsel_s):
    b = pl.program_id(0)
    s = pl.program_id(1)

    @pl.when(s == 0)
    def _route():
        k_ratio = float(D_VIT) / float(D_LLM)
        score = lax.dot_general(
            lq_ref[...], kl_ref[...], (((1,), (1,)), ((), ())),
            preferred_element_type=jnp.float32)
        score = score + k_ratio * lax.dot_general(
            vq_ref[...], kv_ref[...], (((1,), (1,)), ((), ())),
            preferred_element_type=jnp.float32)

        u = lax.bitcast_convert_type(score, jnp.int32)
        key = u ^ ((u >> 31) & jnp.int32(0x7FFFFFFF))

        def count_ge(m):
            return jnp.sum((key >= m).astype(jnp.int32), axis=1,
                           keepdims=True)

        int_min = jnp.full((B, 1), -2147483648, jnp.int32)
        zero = jnp.zeros((B, 1), jnp.int32)
        t = jnp.where(count_ge(zero) >= TOPK, zero, int_min)
        for bit in range(30, -1, -1):
            cand = t | jnp.int32(1 << bit)
            t = jnp.where(count_ge(cand) >= TOPK, cand, t)

        gt = key > t
        eq = key == t
        need = (TOPK - jnp.sum(gt.astype(jnp.int32), axis=1, keepdims=True)
                ).astype(jnp.float32)
        rows = lax.broadcasted_iota(jnp.int32, (K, K), 0)
        cols = lax.broadcasted_iota(jnp.int32, (K, K), 1)
        tri = (rows <= cols).astype(jnp.float32)
        ecs = jnp.dot(eq.astype(jnp.float32), tri,
                      preferred_element_type=jnp.float32)
        mask = jnp.logical_or(gt, jnp.logical_and(eq, ecs <= need))

        mf = mask.astype(jnp.float32)
        pos = jnp.dot(mf, tri, preferred_element_type=jnp.float32)
        rowsel = lax.broadcasted_iota(jnp.int32, (B, 1), 0) == b
        slot_row = (jnp.sum(jnp.where(rowsel, pos, 0.0), axis=0,
                            keepdims=True) - 1.0).astype(jnp.int32)
        mask_row = jnp.sum(jnp.where(rowsel, mf, 0.0), axis=0,
                           keepdims=True)
        jrow = lax.broadcasted_iota(jnp.int32, (TOPK, K), 0)
        ptf = jnp.where((slot_row == jrow) & (mask_row > 0.5), 1.0, 0.0)

        asel_s[...] = lax.dot_general(
            a_ref[...], ptf, (((1,), (1,)), ((), ())),
            preferred_element_type=jnp.float32)
        bsel_s[...] = jnp.dot(ptf, bp_ref[...],
                              preferred_element_type=jnp.float32)

    xb = x_ref[0]
    tt = jnp.dot(xb, asel_s[...], preferred_element_type=jnp.float32)
    out_ref[0] = xb + jnp.dot(tt, bsel_s[...],
                              preferred_element_type=jnp.float32)


@jax.jit
def kernel(x, llm_query, vit_query, static_keys_llm, static_keys_vit,
           rank_A_pool, rank_B_pool):
    return pl.pallas_call(
        _fused_kernel,
        grid=(B, S // TS),
        in_specs=[
            pl.BlockSpec((B, D_LLM), lambda b, s: (0, 0)),
            pl.BlockSpec((B, D_VIT), lambda b, s: (0, 0)),
            pl.BlockSpec((K, D_LLM), lambda b, s: (0, 0)),
            pl.BlockSpec((K, D_VIT), lambda b, s: (0, 0)),
            pl.BlockSpec((D_LLM, K), lambda b, s: (0, 0)),
            pl.BlockSpec((K, D_LLM), lambda b, s: (0, 0)),
            pl.BlockSpec((1, TS, D_LLM), lambda b, s: (b, s, 0)),
        ],
        out_specs=pl.BlockSpec((1, TS, D_LLM), lambda b, s: (b, s, 0)),
        out_shape=jax.ShapeDtypeStruct((B, S, D_LLM), jnp.float32),
        scratch_shapes=[
            pltpu.VMEM((D_LLM, TOPK), jnp.float32),
            pltpu.VMEM((TOPK, D_LLM), jnp.float32),
        ],
    )(llm_query, vit_query, static_keys_llm, static_keys_vit,
      rank_A_pool, rank_B_pool, x)


# speedup vs baseline: 1.5348x; 1.3497x over previous
import jax
import jax.numpy as jnp
from jax import lax
from jax.experimental import pallas as pl
from jax.experimental.pallas import tpu as pltpu

B, S, D_LLM, D_VIT, K, TOPK = 4, 2048, 2048, 1024, 512, 128
TS = 512


def _fused_kernel(lq_ref, vq_ref, kl_ref, kv_ref, a_ref, bp_ref, x_ref,
                  out_ref, mf_s, pos_s, asel_s, bsel_s):
    b = pl.program_id(0)
    s = pl.program_id(1)

    @pl.when((b == 0) & (s == 0))
    def _route():
        k_ratio = float(D_VIT) / float(D_LLM)
        score = lax.dot_general(
            lq_ref[...], kl_ref[...], (((1,), (1,)), ((), ())),
            preferred_element_type=jnp.float32)
        score = score + k_ratio * lax.dot_general(
            vq_ref[...], kv_ref[...], (((1,), (1,)), ((), ())),
            preferred_element_type=jnp.float32)

        u = lax.bitcast_convert_type(score, jnp.int32)
        key = u ^ ((u >> 31) & jnp.int32(0x7FFFFFFF))

        def count_ge(m):
            return jnp.sum((key >= m).astype(jnp.int32), axis=1,
                           keepdims=True)

        int_min = jnp.full((B, 1), -2147483648, jnp.int32)
        zero = jnp.zeros((B, 1), jnp.int32)
        t = jnp.where(count_ge(zero) >= TOPK, zero, int_min)
        for bit in range(30, -1, -1):
            cand = t | jnp.int32(1 << bit)
            t = jnp.where(count_ge(cand) >= TOPK, cand, t)

        gt = key > t
        eq = key == t
        need = (TOPK - jnp.sum(gt.astype(jnp.int32), axis=1, keepdims=True)
                ).astype(jnp.float32)
        rows = lax.broadcasted_iota(jnp.int32, (K, K), 0)
        cols = lax.broadcasted_iota(jnp.int32, (K, K), 1)
        tri = (rows <= cols).astype(jnp.float32)
        ecs = jnp.dot(eq.astype(jnp.float32), tri,
                      preferred_element_type=jnp.float32)
        mask = jnp.logical_or(gt, jnp.logical_and(eq, ecs <= need))

        mf = mask.astype(jnp.float32)
        mf_s[...] = mf
        pos_s[...] = jnp.dot(mf, tri, preferred_element_type=jnp.float32)

    @pl.when(s == 0)
    def _select():
        rowsel = lax.broadcasted_iota(jnp.int32, (B, 1), 0) == b
        slot_row = (jnp.sum(jnp.where(rowsel, pos_s[...], 0.0), axis=0,
                            keepdims=True) - 1.0).astype(jnp.int32)
        mask_row = jnp.sum(jnp.where(rowsel, mf_s[...], 0.0), axis=0,
                           keepdims=True)
        jrow = lax.broadcasted_iota(jnp.int32, (TOPK, K), 0)
        ptf = jnp.where((slot_row == jrow) & (mask_row > 0.5), 1.0, 0.0)

        asel_s[...] = lax.dot_general(
            a_ref[...], ptf, (((1,), (1,)), ((), ())),
            preferred_element_type=jnp.float32)
        bsel_s[...] = jnp.dot(ptf, bp_ref[...],
                              preferred_element_type=jnp.float32)

    xb = x_ref[0]
    tt = jnp.dot(xb, asel_s[...], preferred_element_type=jnp.float32)
    out_ref[0] = xb + jnp.dot(tt, bsel_s[...],
                              preferred_element_type=jnp.float32)


@jax.jit
def kernel(x, llm_query, vit_query, static_keys_llm, static_keys_vit,
           rank_A_pool, rank_B_pool):
    return pl.pallas_call(
        _fused_kernel,
        grid=(B, S // TS),
        in_specs=[
            pl.BlockSpec((B, D_LLM), lambda b, s: (0, 0)),
            pl.BlockSpec((B, D_VIT), lambda b, s: (0, 0)),
            pl.BlockSpec((K, D_LLM), lambda b, s: (0, 0)),
            pl.BlockSpec((K, D_VIT), lambda b, s: (0, 0)),
            pl.BlockSpec((D_LLM, K), lambda b, s: (0, 0)),
            pl.BlockSpec((K, D_LLM), lambda b, s: (0, 0)),
            pl.BlockSpec((1, TS, D_LLM), lambda b, s: (b, s, 0)),
        ],
        out_specs=pl.BlockSpec((1, TS, D_LLM), lambda b, s: (b, s, 0)),
        out_shape=jax.ShapeDtypeStruct((B, S, D_LLM), jnp.float32),
        scratch_shapes=[
            pltpu.VMEM((B, K), jnp.float32),
            pltpu.VMEM((B, K), jnp.float32),
            pltpu.VMEM((D_LLM, TOPK), jnp.float32),
            pltpu.VMEM((TOPK, D_LLM), jnp.float32),
        ],
    )(llm_query, vit_query, static_keys_llm, static_keys_vit,
      rank_A_pool, rank_B_pool, x)


# speedup vs baseline: 1.6172x; 1.0537x over previous
import jax
import jax.numpy as jnp
from jax import lax
from jax.experimental import pallas as pl
from jax.experimental.pallas import tpu as pltpu

B, S, D_LLM, D_VIT, K, TOPK = 4, 2048, 2048, 1024, 512, 128
TS = 1024


def _fused_kernel(lq_ref, vq_ref, kl_ref, kv_ref, a_ref, bp_ref, x_ref,
                  out_ref, mf_s, pos_s, asel_s, bsel_s):
    b = pl.program_id(0)
    s = pl.program_id(1)

    @pl.when((b == 0) & (s == 0))
    def _route():
        k_ratio = float(D_VIT) / float(D_LLM)
        score = lax.dot_general(
            lq_ref[...], kl_ref[...], (((1,), (1,)), ((), ())),
            preferred_element_type=jnp.float32)
        score = score + k_ratio * lax.dot_general(
            vq_ref[...], kv_ref[...], (((1,), (1,)), ((), ())),
            preferred_element_type=jnp.float32)

        u = lax.bitcast_convert_type(score, jnp.int32)
        key = u ^ ((u >> 31) & jnp.int32(0x7FFFFFFF))

        def count_ge(m):
            return jnp.sum((key >= m).astype(jnp.int32), axis=1,
                           keepdims=True)

        int_min = jnp.full((B, 1), -2147483648, jnp.int32)
        zero = jnp.zeros((B, 1), jnp.int32)
        t = jnp.where(count_ge(zero) >= TOPK, zero, int_min)
        for bit in range(30, -1, -1):
            cand = t | jnp.int32(1 << bit)
            t = jnp.where(count_ge(cand) >= TOPK, cand, t)

        gt = key > t
        eq = key == t
        need = (TOPK - jnp.sum(gt.astype(jnp.int32), axis=1, keepdims=True)
                ).astype(jnp.float32)
        rows = lax.broadcasted_iota(jnp.int32, (K, K), 0)
        cols = lax.broadcasted_iota(jnp.int32, (K, K), 1)
        tri = (rows <= cols).astype(jnp.float32)
        ecs = jnp.dot(eq.astype(jnp.float32), tri,
                      preferred_element_type=jnp.float32)
        mask = jnp.logical_or(gt, jnp.logical_and(eq, ecs <= need))

        mf = mask.astype(jnp.float32)
        mf_s[...] = mf
        pos_s[...] = jnp.dot(mf, tri, preferred_element_type=jnp.float32)

    @pl.when(s == 0)
    def _select():
        rowsel = lax.broadcasted_iota(jnp.int32, (B, 1), 0) == b
        slot_row = (jnp.sum(jnp.where(rowsel, pos_s[...], 0.0), axis=0,
                            keepdims=True) - 1.0).astype(jnp.int32)
        mask_row = jnp.sum(jnp.where(rowsel, mf_s[...], 0.0), axis=0,
                           keepdims=True)
        jrow = lax.broadcasted_iota(jnp.int32, (TOPK, K), 0)
        ptf = jnp.where((slot_row == jrow) & (mask_row > 0.5), 1.0, 0.0)

        asel_s[...] = lax.dot_general(
            a_ref[...], ptf, (((1,), (1,)), ((), ())),
            preferred_element_type=jnp.float32)
        bsel_s[...] = jnp.dot(ptf, bp_ref[...],
                              preferred_element_type=jnp.float32)

    xb = x_ref[0]
    tt = jnp.dot(xb, asel_s[...], preferred_element_type=jnp.float32)
    out_ref[0] = xb + jnp.dot(tt, bsel_s[...],
                              preferred_element_type=jnp.float32)


@jax.jit
def kernel(x, llm_query, vit_query, static_keys_llm, static_keys_vit,
           rank_A_pool, rank_B_pool):
    return pl.pallas_call(
        _fused_kernel,
        grid=(B, S // TS),
        in_specs=[
            pl.BlockSpec((B, D_LLM), lambda b, s: (0, 0)),
            pl.BlockSpec((B, D_VIT), lambda b, s: (0, 0)),
            pl.BlockSpec((K, D_LLM), lambda b, s: (0, 0)),
            pl.BlockSpec((K, D_VIT), lambda b, s: (0, 0)),
            pl.BlockSpec((D_LLM, K), lambda b, s: (0, 0)),
            pl.BlockSpec((K, D_LLM), lambda b, s: (0, 0)),
            pl.BlockSpec((1, TS, D_LLM), lambda b, s: (b, s, 0)),
        ],
        out_specs=pl.BlockSpec((1, TS, D_LLM), lambda b, s: (b, s, 0)),
        out_shape=jax.ShapeDtypeStruct((B, S, D_LLM), jnp.float32),
        scratch_shapes=[
            pltpu.VMEM((B, K), jnp.float32),
            pltpu.VMEM((B, K), jnp.float32),
            pltpu.VMEM((D_LLM, TOPK), jnp.float32),
            pltpu.VMEM((TOPK, D_LLM), jnp.float32),
        ],
    )(llm_query, vit_query, static_keys_llm, static_keys_vit,
      rank_A_pool, rank_B_pool, x)
